# baseline (speedup 1.0000x reference)
import functools

import jax
import jax.numpy as jnp
from jax import lax
from jax.experimental import pallas as pl
from jax.experimental.pallas import tpu as pltpu
from jax.experimental.pallas import tpu_sc as plsc

_D = 768
_ROWS = 576
_BPW = 24
_NW_ACTIVE = _ROWS // _BPW
_LANES = 16


@functools.partial(
    pl.kernel,
    out_type=jax.ShapeDtypeStruct(((_ROWS + 1) * _D,), jnp.float32),
    mesh=plsc.VectorSubcoreMesh(core_axis_name="c", subcore_axis_name="s"),
    scratch_types=[
        pltpu.VMEM((_BPW,), jnp.int32),
        pltpu.VMEM((_BPW, _D), jnp.float32),
        pltpu.VMEM((_D,), jnp.float32),
        pltpu.SemaphoreType.DMA,
    ],
)
def _emb_pad_kernel(idx_hbm, table_hbm, out_hbm, idx_v, rows_v, zrow_v, sem):
    wid = lax.axis_index("s") * 2 + lax.axis_index("c")

    @pl.when(wid < _NW_ACTIVE)
    def _gather_scatter():
        base = wid * _BPW
        pltpu.sync_copy(idx_hbm.at[pl.ds(base, _BPW)], idx_v)
        pltpu.async_copy(table_hbm.at[idx_v], rows_v, sem).wait()
        copies = [
            pltpu.async_copy(
                rows_v.at[i], out_hbm.at[pl.ds((base + 1 + i) * _D, _D)], sem)
            for i in range(_BPW)
        ]
        for c in copies:
            c.wait()

    @pl.when(wid == _NW_ACTIVE)
    def _zero_row():
        zeros = jnp.zeros((_LANES,), jnp.float32)
        for j in range(_D // _LANES):
            zrow_v[pl.ds(j * _LANES, _LANES)] = zeros
        pltpu.sync_copy(zrow_v, out_hbm.at[pl.ds(0, _D)])


def kernel(x, spatial_indices_sequence, emb_weight):
    del x
    idx = spatial_indices_sequence.astype(jnp.int32)
    out = _emb_pad_kernel(idx, emb_weight)
    return out.reshape(1, _ROWS + 1, _D)


# speedup vs baseline: 1.0303x; 1.0303x over previous
import functools

import jax
import jax.numpy as jnp
from jax import lax
from jax.experimental import pallas as pl
from jax.experimental.pallas import tpu as pltpu
from jax.experimental.pallas import tpu_sc as plsc

_D = 768
_ROWS = 576
_BPW = 24
_NW_ACTIVE = _ROWS // _BPW
_LANES = 16


@functools.partial(
    pl.kernel,
    out_type=jax.ShapeDtypeStruct((_ROWS + 1, _D), jnp.float32),
    mesh=plsc.VectorSubcoreMesh(core_axis_name="c", subcore_axis_name="s"),
    compiler_params=pltpu.CompilerParams(use_tc_tiling_on_sc=False),
    scratch_types=[
        pltpu.VMEM((_BPW,), jnp.int32),
        pltpu.VMEM((_BPW, _D), jnp.float32),
        pltpu.VMEM((_D,), jnp.float32),
        pltpu.SemaphoreType.DMA,
    ],
)
def _emb_pad_kernel(idx_hbm, table_hbm, out_hbm, idx_v, rows_v, zrow_v, sem):
    wid = lax.axis_index("s") * 2 + lax.axis_index("c")

    @pl.when(wid < _NW_ACTIVE)
    def _gather_scatter():
        base = wid * _BPW
        pltpu.sync_copy(idx_hbm.at[pl.ds(base, _BPW)], idx_v)
        pltpu.async_copy(table_hbm.at[idx_v], rows_v, sem).wait()
        pltpu.sync_copy(rows_v, out_hbm.at[pl.ds(base + 1, _BPW)])

    @pl.when(wid == _NW_ACTIVE)
    def _zero_row():
        zeros = jnp.zeros((_LANES,), jnp.float32)
        for j in range(_D // _LANES):
            zrow_v[pl.ds(j * _LANES, _LANES)] = zeros
        pltpu.sync_copy(zrow_v, out_hbm.at[0])


def kernel(x, spatial_indices_sequence, emb_weight):
    del x
    idx = spatial_indices_sequence.astype(jnp.int32)
    out = _emb_pad_kernel(idx, emb_weight)
    return out[None]


# speedup vs baseline: 1.0950x; 1.0628x over previous
import functools

import jax
import jax.numpy as jnp
from jax import lax
from jax.experimental import pallas as pl
from jax.experimental.pallas import tpu as pltpu
from jax.experimental.pallas import tpu_sc as plsc

_D = 768
_ROWS = 576
_BPW = 24
_NW_ACTIVE = _ROWS // _BPW
_LANES = 16
_WAVE = 8
_NWAVES = _BPW // _WAVE


@functools.partial(
    pl.kernel,
    out_type=jax.ShapeDtypeStruct((1, _ROWS + 1, _D), jnp.float32),
    mesh=plsc.VectorSubcoreMesh(core_axis_name="c", subcore_axis_name="s"),
    scratch_types=[
        pltpu.VMEM((_BPW,), jnp.int32),
        pltpu.VMEM((_BPW,), jnp.int32),
        pltpu.VMEM((_BPW, _D), jnp.float32),
        pltpu.VMEM((1, _D), jnp.float32),
        pltpu.SemaphoreType.DMA,
        pltpu.SemaphoreType.DMA,
    ],
)
def _emb_pad_kernel(idx_hbm, table_hbm, out_hbm, idx_v, oidx_v, rows_v,
                    zrow_v, gsem, ssem):
    wid = lax.axis_index("s") * 2 + lax.axis_index("c")

    @pl.when(wid < _NW_ACTIVE)
    def _gather_scatter():
        base = wid * _BPW
        row0 = base + 1
        oidx_v[pl.ds(0, _LANES)] = row0 + lax.iota(jnp.int32, _LANES)
        oidx_v[pl.ds(_BPW - _LANES, _LANES)] = (
            row0 + (_BPW - _LANES) + lax.iota(jnp.int32, _LANES))
        pltpu.sync_copy(idx_hbm.at[pl.ds(base, _BPW)], idx_v)
        out2d = out_hbm.at[0]
        gathers = [
            pltpu.async_copy(
                table_hbm.at[idx_v.at[pl.ds(w * _WAVE, _WAVE)]],
                rows_v.at[pl.ds(w * _WAVE, _WAVE)], gsem)
            for w in range(_NWAVES)
        ]
        scatters = []
        for w in range(_NWAVES):
            gathers[w].wait()
            scatters.append(pltpu.async_copy(
                rows_v.at[pl.ds(w * _WAVE, _WAVE)],
                out2d.at[oidx_v.at[pl.ds(w * _WAVE, _WAVE)]], ssem))
        for s in scatters:
            s.wait()

    @pl.when(wid == _NW_ACTIVE)
    def _zero_row():
        zeros = jnp.zeros((_LANES,), jnp.float32)
        for j in range(_D // _LANES):
            zrow_v[0, pl.ds(j * _LANES, _LANES)] = zeros
        pltpu.sync_copy(zrow_v, out_hbm.at[0].at[pl.ds(0, 1)])


def kernel(x, spatial_indices_sequence, emb_weight):
    del x
    idx = spatial_indices_sequence.astype(jnp.int32)
    return _emb_pad_kernel(idx, emb_weight)
